# Initial kernel scaffold; baseline (speedup 1.0000x reference)
import jax
import jax.numpy as jnp
from jax.experimental import pallas as pl


def kernel(x, params, edge_index, batch):
    raise NotImplementedError("write your pallas kernel here")


# speedup vs baseline: 1.0379x; 1.0379x over previous
import functools

import jax
import jax.numpy as jnp
import numpy as np
from jax.experimental import pallas as pl
from jax.experimental.pallas import tpu as pltpu

N_NODES = 10000
B_GRAPHS = 64
RATIO = 0.8


def _mm_body(x_ref, w_ref, b_ref, o_ref, *, act):
    acc = jnp.dot(x_ref[...], w_ref[...], preferred_element_type=jnp.float32)
    acc = acc + b_ref[...]
    if act == "relu":
        acc = jnp.maximum(acc, 0.0)
    o_ref[...] = acc


def _matmul_bias(x, w, b, act="none", bm=512):
    m, k = x.shape
    k2, n = w.shape
    assert k == k2
    mp = ((m + bm - 1) // bm) * bm
    if mp != m:
        x = jnp.pad(x, ((0, mp - m), (0, 0)))
    out = pl.pallas_call(
        functools.partial(_mm_body, act=act),
        grid=(mp // bm,),
        in_specs=[
            pl.BlockSpec((bm, k), lambda i: (i, 0)),
            pl.BlockSpec((k, n), lambda i: (0, 0)),
            pl.BlockSpec((1, n), lambda i: (0, 0)),
        ],
        out_specs=pl.BlockSpec((bm, n), lambda i: (i, 0)),
        out_shape=jax.ShapeDtypeStruct((mp, n), jnp.float32),
    )(x, w, b.reshape(1, n))
    return out[:m]


def _gcn(x, edge_index, W, b):
    N = x.shape[0]
    row = edge_index[0]
    col = edge_index[1]
    sl = jnp.arange(N, dtype=row.dtype)
    row = jnp.concatenate([row, sl])
    col = jnp.concatenate([col, sl])
    h = _matmul_bias(x, W, jnp.zeros((W.shape[1],), jnp.float32))
    deg = jax.ops.segment_sum(jnp.ones_like(col, dtype=h.dtype), col, num_segments=N)
    dinv = jnp.where(deg > 0, jax.lax.rsqrt(jnp.maximum(deg, 1e-12)), 0.0)
    coef = dinv[row] * dinv[col]
    out = jax.ops.segment_sum(h[row] * coef[:, None], col, num_segments=N)
    return out + b


def _block(x, edge_index, W1, b1, W2, b2, Wl, bl):
    x1 = jax.nn.relu(_gcn(x, edge_index, W1, b1))
    x2 = jax.nn.relu(_gcn(x1, edge_index, W2, b2))
    return _matmul_bias(jnp.concatenate([x1, x2], axis=1), Wl, bl)


def _seg_max_safe(x, seg, num):
    m = jax.ops.segment_max(x, seg, num_segments=num)
    return jnp.where(jnp.isfinite(m), m, 0.0)


def _topk_select(score, batch, num_graphs, ratio):
    N = score.shape[0]
    counts = jnp.bincount(batch, length=num_graphs)
    k = jnp.ceil(ratio * counts.astype(jnp.float32)).astype(jnp.int32)
    key = batch.astype(jnp.float32) * 4.0 - score
    perm = jnp.argsort(key)
    sb = batch[perm]
    starts = jnp.concatenate([jnp.zeros((1,), counts.dtype), jnp.cumsum(counts)[:-1]])
    rank = jnp.arange(N, dtype=jnp.int32) - starts[sb].astype(jnp.int32)
    keep = rank < k[sb]
    pos = jnp.cumsum(keep) - 1
    kept_full = jnp.full((N,), N, dtype=perm.dtype).at[
        jnp.where(keep, pos, N)].set(perm, mode='drop')
    return kept_full, jnp.sum(keep)


def _filter_adj(edge_index, kept, N):
    node_mask = jnp.zeros((N,), dtype=bool).at[kept].set(True, mode='drop')
    n_idx = jnp.full((N,), -1, dtype=edge_index.dtype).at[kept].set(
        jnp.arange(kept.shape[0], dtype=edge_index.dtype), mode='drop')
    row, col = edge_index[0], edge_index[1]
    emask = node_mask[row] & node_mask[col]
    new_row = jnp.where(emask, n_idx[row], 0)
    new_col = jnp.where(emask, n_idx[col], N)
    return jnp.stack([new_row, new_col])


def kernel(x, params, edge_index, batch):
    p = params
    x = jax.nn.relu(_block(x, edge_index, p['b1_c1_W'], p['b1_c1_b'],
                           p['b1_c2_W'], p['b1_c2_b'], p['b1_lin_W'], p['b1_lin_b']))
    xs = [jax.ops.segment_sum(x, batch, num_segments=B_GRAPHS),
          _seg_max_safe(x, batch, B_GRAPHS)]
    w = p['pool_p']
    score = jnp.tanh((x @ w) / jnp.linalg.norm(w))
    kept, nkept = _topk_select(score, batch, B_GRAPHS, RATIO)
    ei2 = _filter_adj(edge_index, kept, x.shape[0])
    valid = jnp.arange(x.shape[0], dtype=nkept.dtype) < nkept
    batch2 = jnp.where(valid, batch[kept], B_GRAPHS)
    x = jnp.where(valid[:, None], x[kept] * score[kept][:, None], 0.0)
    x = jax.nn.relu(_block(x, ei2, p['b2_c1_W'], p['b2_c1_b'],
                           p['b2_c2_W'], p['b2_c2_b'], p['b2_lin_W'], p['b2_lin_b']))
    xs.extend([jax.ops.segment_sum(x, batch2, num_segments=B_GRAPHS),
               _seg_max_safe(x, batch2, B_GRAPHS)])
    h = jnp.concatenate(xs, axis=1)
    h = jax.nn.relu(_matmul_bias(h, p['lin1_W'], p['lin1_b'], bm=64))
    out = jax.nn.softmax(_matmul_bias(h, p['lin2_W'], p['lin2_b'], bm=64), axis=-1)
    return out


# speedup vs baseline: 4.1540x; 4.0021x over previous
import functools

import jax
import jax.numpy as jnp
import numpy as np
from jax import lax
from jax.experimental import pallas as pl
from jax.experimental.pallas import tpu as pltpu
from jax.experimental.pallas import tpu_sc as plsc

N = 10000
NP = 10240
F = 256
B = 64
NB = 16
BASE = N // NB
C = 128
E = 320000
EP = E + NB * C
TB = 20016
DUMMY = 10000
SLAB = BASE * 128

_mesh = plsc.VectorSubcoreMesh(core_axis_name="c", subcore_axis_name="s")
_scparams = pltpu.CompilerParams(needs_layout_passes=False)


def _mm_body(x_ref, w_ref, b_ref, pre_ref, deg_ref, o_ref, *, act, use_pre, use_deg):
    x = x_ref[...]
    if use_pre:
        x = x * pre_ref[...]
    acc = jnp.dot(x, w_ref[...], preferred_element_type=jnp.float32)
    acc = acc + b_ref[...]
    if act == "relu":
        acc = jnp.maximum(acc, 0.0)
    elif act == "softmax":
        col = lax.broadcasted_iota(jnp.int32, acc.shape, 1)
        acc = jnp.where(col < 10, acc, -1e30)
        acc = acc - jnp.max(acc, axis=1, keepdims=True)
        e = jnp.exp(acc)
        acc = e / jnp.sum(e, axis=1, keepdims=True)
    if use_deg:
        acc = acc * lax.rsqrt(jnp.maximum(deg_ref[...], 1e-12))
    o_ref[...] = acc


def _mm(x, w, b, act="none", pre=None, deg=None, bm=512):
    m, k = x.shape
    n = w.shape[1]
    use_pre = pre is not None
    use_deg = deg is not None
    if pre is None:
        pre = jnp.zeros((m, 1), jnp.float32)
    if deg is None:
        deg = jnp.ones((m, 1), jnp.float32)
    out = pl.pallas_call(
        functools.partial(_mm_body, act=act, use_pre=use_pre, use_deg=use_deg),
        grid=(m // bm,),
        in_specs=[
            pl.BlockSpec((bm, k), lambda i: (i, 0)),
            pl.BlockSpec((k, n), lambda i: (0, 0)),
            pl.BlockSpec((1, n), lambda i: (0, 0)),
            pl.BlockSpec((bm, 1), lambda i: (i, 0)),
            pl.BlockSpec((bm, 1), lambda i: (i, 0)),
        ],
        out_specs=pl.BlockSpec((bm, n), lambda i: (i, 0)),
        out_shape=jax.ShapeDtypeStruct((m, n), jnp.float32),
    )(x, w, b.reshape(1, n), pre.reshape(m, 1), deg.reshape(m, 1))
    return out


def _post_body(s_ref, g_ref, deg_ref, b_ref, o_ref):
    dinv = lax.rsqrt(jnp.maximum(deg_ref[...], 1e-12))
    o_ref[...] = jnp.maximum(dinv * (s_ref[...] + g_ref[...]) + b_ref[...], 0.0)


def _gcn_post(s, g, deg, b, bm=512):
    m = s.shape[0]
    return pl.pallas_call(
        _post_body,
        grid=(m // bm,),
        in_specs=[
            pl.BlockSpec((bm, F), lambda i: (i, 0)),
            pl.BlockSpec((bm, F), lambda i: (i, 0)),
            pl.BlockSpec((bm, 1), lambda i: (i, 0)),
            pl.BlockSpec((1, F), lambda i: (0, 0)),
        ],
        out_specs=pl.BlockSpec((bm, F), lambda i: (i, 0)),
        out_shape=jax.ShapeDtypeStruct((m, F), jnp.float32),
    )(s, g, deg.reshape(m, 1), b.reshape(1, F))


def _pool_body(x_ref, bat_ref, w_ref, sum_ref, max_ref, cnt_ref, sc_ref):
    i = pl.program_id(0)

    @pl.when(i == 0)
    def _():
        sum_ref[...] = jnp.zeros_like(sum_ref)
        max_ref[...] = jnp.full_like(max_ref, -jnp.inf)
        cnt_ref[...] = jnp.zeros_like(cnt_ref)

    x = x_ref[...]
    bat = bat_ref[...]
    wp = w_ref[...]
    nrm = jnp.sqrt(jnp.sum(wp * wp))
    sc_ref[...] = jnp.tanh(jnp.dot(x, wp, preferred_element_type=jnp.float32) / nrm)

    gi = lax.broadcasted_iota(jnp.int32, (x.shape[0], B), 1)
    oh = (bat == gi).astype(jnp.float32)
    sum_ref[...] += lax.dot_general(oh, x, (((0,), (0,)), ((), ())),
                                   preferred_element_type=jnp.float32)
    cnt_ref[...] += jnp.sum(oh, axis=0)[:, None]

    def body(g, _):
        mask = bat == g
        mx = jnp.max(jnp.where(mask, x, -jnp.inf), axis=0, keepdims=True)
        max_ref[pl.ds(g, 1), :] = jnp.maximum(max_ref[pl.ds(g, 1), :], mx)
        return 0
    lax.fori_loop(0, B, body, 0)

    @pl.when(i == pl.num_programs(0) - 1)
    def _():
        m = max_ref[...]
        max_ref[...] = jnp.where(jnp.isfinite(m), m, 0.0)


def _pools(x, bat, wp, bm=512):
    m = x.shape[0]
    return pl.pallas_call(
        _pool_body,
        grid=(m // bm,),
        in_specs=[
            pl.BlockSpec((bm, F), lambda i: (i, 0)),
            pl.BlockSpec((bm, 1), lambda i: (i, 0)),
            pl.BlockSpec((F, 128), lambda i: (0, 0)),
        ],
        out_specs=[
            pl.BlockSpec((B, F), lambda i: (0, 0)),
            pl.BlockSpec((B, F), lambda i: (0, 0)),
            pl.BlockSpec((B, 128), lambda i: (0, 0)),
            pl.BlockSpec((bm, 128), lambda i: (i, 0)),
        ],
        out_shape=[
            jax.ShapeDtypeStruct((B, F), jnp.float32),
            jax.ShapeDtypeStruct((B, F), jnp.float32),
            jax.ShapeDtypeStruct((B, 128), jnp.float32),
            jax.ShapeDtypeStruct((m, 128), jnp.float32),
        ],
    )(x, bat.reshape(m, 1), wp)


def _topk_body(s_ref, bat_ref, cnt_ref, ks_ref, kf_ref, b2_ref, t_s, t2_s, k_s, m_s):
    score = s_ref[...]
    bat = bat_ref[...]
    batf = bat.astype(jnp.float32)
    val = -(batf * 4.0 - score)
    u = lax.bitcast_convert_type(val, jnp.int32)
    sk = jnp.where(u < 0, u ^ np.int32(0x7FFFFFFF), u)
    msb = np.int32(-(2 ** 31))
    ss = sk ^ msb
    valid = bat < B

    def kinit(g, _):
        c = cnt_ref[pl.ds(g, 1), pl.ds(0, 1)][0, 0].astype(jnp.int32)
        k_s[g] = (4 * c + 4) // 5
        t_s[g] = 0
        t2_s[g] = 0
        return 0
    lax.fori_loop(0, B, kinit, 0)

    bits = [np.int32(-(2 ** 31))] + [np.int32(1 << b) for b in range(30, -1, -1)]
    for bit in bits:
        def gbody(g, _):
            cand = t_s[g] | bit
            scand = cand ^ msb
            ind = (ss >= scand) & (bat == g)
            cnt = jnp.sum(ind.astype(jnp.float32))
            t_s[g] = jnp.where(cnt >= k_s[g].astype(jnp.float32), cand, t_s[g])
            return 0
        lax.fori_loop(0, B, gbody, 0)

    def thr_body(g, acc):
        return jnp.where(bat == g, t_s[g], acc)
    thr = lax.fori_loop(0, B, thr_body, jnp.zeros_like(ss))
    sthr = thr ^ msb
    gt = ss > sthr
    tie = (ss == sthr) & valid

    def mb(g, _):
        ngt = jnp.sum((gt & (bat == g)).astype(jnp.float32))
        m_s[g] = k_s[g] - ngt.astype(jnp.int32)
        return 0
    lax.fori_loop(0, B, mb, 0)

    ridx = np.int32(16383) - (lax.broadcasted_iota(jnp.int32, score.shape, 0) * 128
                              + lax.broadcasted_iota(jnp.int32, score.shape, 1))
    for bit in [np.int32(1 << b) for b in range(13, -1, -1)]:
        def g2body(g, _):
            cand = t2_s[g] | bit
            ind = tie & (ridx >= cand) & (bat == g)
            cnt = jnp.sum(ind.astype(jnp.float32))
            t2_s[g] = jnp.where(cnt >= m_s[g].astype(jnp.float32), cand, t2_s[g])
            return 0
        lax.fori_loop(0, B, g2body, 0)

    def thr2_body(g, acc):
        return jnp.where(bat == g, t2_s[g], acc)
    thr2 = lax.fori_loop(0, B, thr2_body, jnp.zeros_like(ss))
    keep = (gt | (tie & (ridx >= thr2))) & valid
    keepf = keep.astype(jnp.float32)
    kf_ref[...] = keepf
    ks_ref[...] = keepf * score
    b2_ref[...] = jnp.where(keep, bat, B)


def _topk(s80, b80, cnt):
    return pl.pallas_call(
        _topk_body,
        grid=(1,),
        in_specs=[
            pl.BlockSpec((NP // 128, 128), lambda i: (0, 0)),
            pl.BlockSpec((NP // 128, 128), lambda i: (0, 0)),
            pl.BlockSpec((B, 128), lambda i: (0, 0)),
        ],
        out_specs=[
            pl.BlockSpec((NP // 128, 128), lambda i: (0, 0)),
            pl.BlockSpec((NP // 128, 128), lambda i: (0, 0)),
            pl.BlockSpec((NP // 128, 128), lambda i: (0, 0)),
        ],
        out_shape=[
            jax.ShapeDtypeStruct((NP // 128, 128), jnp.float32),
            jax.ShapeDtypeStruct((NP // 128, 128), jnp.float32),
            jax.ShapeDtypeStruct((NP // 128, 128), jnp.int32),
        ],
        scratch_shapes=[pltpu.SMEM((B,), jnp.int32)] * 4,
    )(s80, b80, cnt)


@functools.partial(
    pl.kernel, mesh=_mesh, compiler_params=_scparams,
    out_type=jax.ShapeDtypeStruct((32, SLAB), jnp.float32),
    scratch_types=[
        pltpu.VMEM((16,), jnp.int32),
        pltpu.VMEM((16,), jnp.int32),
        pltpu.VMEM((C,), jnp.int32),
        pltpu.VMEM((C,), jnp.int32),
        pltpu.VMEM((C, 128), jnp.float32),
        pltpu.VMEM((SLAB,), jnp.float32),
        pltpu.SemaphoreType.DMA,
    ],
)
def _sc_conv(tab_hbm, row0_hbm, row1_hbm, col_hbm, bs_hbm, nc_hbm, out_hbm,
             bsv, ncv, rowv, colv, buf, slab, gsem):
    tid = lax.axis_index("s")
    cid = lax.axis_index("c")
    IOTA = lax.iota(jnp.int32, 16)

    pltpu.sync_copy(bs_hbm, bsv)
    pltpu.sync_copy(nc_hbm, ncv)
    lo = jnp.sum(jnp.where(IOTA == tid, bsv[...], 0))
    nch = jnp.sum(jnp.where(IOTA == tid, ncv[...], 0))

    zero = jnp.zeros((16,), jnp.float32)

    def zbody(i, _):
        for j in range(8):
            slab[pl.ds(i * 128 + j * 16, 16)] = zero
        return 0
    lax.fori_loop(0, SLAB // 128, zbody, 0)

    def body(i, _):
        base = pl.multiple_of(lo + i * C, 128)

        @pl.when(cid == 0)
        def _():
            pltpu.sync_copy(row0_hbm.at[pl.ds(base, C)], rowv)

        @pl.when(cid == 1)
        def _():
            pltpu.sync_copy(row1_hbm.at[pl.ds(base, C)], rowv)
        pltpu.sync_copy(col_hbm.at[pl.ds(base, C)], colv)
        pltpu.async_copy(tab_hbm.at[rowv], buf, gsem).wait()

        def ebody(e, _):
            for u in range(2):
                ei = e * 2 + u
                cvec = plsc.load_gather(colv, [jnp.full((16,), ei, jnp.int32)])
                a0 = cvec * 128 + IOTA
                for kk in range(8):
                    dat = buf[ei, pl.ds(kk * 16, 16)]
                    plsc.addupdate_scatter(slab, [a0 + (kk * 16)], dat)
            return 0
        lax.fori_loop(0, C // 2, ebody, 0)
        return 0

    lax.fori_loop(0, nch, body, 0)
    pltpu.sync_copy(slab, out_hbm.at[cid * 16 + tid])


@functools.partial(
    pl.kernel, mesh=_mesh, compiler_params=_scparams,
    out_type=jax.ShapeDtypeStruct((16, 640), jnp.float32),
    scratch_types=[
        pltpu.VMEM((16,), jnp.int32),
        pltpu.VMEM((16,), jnp.int32),
        pltpu.VMEM((10016,), jnp.float32),
        pltpu.VMEM((C,), jnp.int32),
        pltpu.VMEM((C,), jnp.int32),
        pltpu.VMEM((640,), jnp.float32),
    ],
)
def _sc_deg(keep_hbm, row_hbm, col_hbm, bs_hbm, nc_hbm, out_hbm,
            bsv, ncv, keepv, rowv, colv, slab):
    tid = lax.axis_index("s")
    cid = lax.axis_index("c")
    IOTA = lax.iota(jnp.int32, 16)

    @pl.when(cid == 0)
    def _():
        pltpu.sync_copy(bs_hbm, bsv)
        pltpu.sync_copy(nc_hbm, ncv)
        pltpu.sync_copy(keep_hbm, keepv)
        lo = jnp.sum(jnp.where(IOTA == tid, bsv[...], 0))
        nch = jnp.sum(jnp.where(IOTA == tid, ncv[...], 0))
        zero = jnp.zeros((16,), jnp.float32)
        for i in range(640 // 16):
            slab[pl.ds(i * 16, 16)] = zero

        def body(i, _):
            base = pl.multiple_of(lo + i * C, 128)
            pltpu.sync_copy(row_hbm.at[pl.ds(base, C)], rowv)
            pltpu.sync_copy(col_hbm.at[pl.ds(base, C)], colv)

            def ebody(j, _):
                for u in range(2):
                    jj = (j * 2 + u) * 16
                    kv = plsc.load_gather(keepv, [rowv[pl.ds(jj, 16)]])
                    plsc.addupdate_scatter(slab, [colv[pl.ds(jj, 16)]], kv)
                return 0
            lax.fori_loop(0, C // 32, ebody, 0)
            return 0

        lax.fori_loop(0, nch, body, 0)
        pltpu.sync_copy(slab, out_hbm.at[tid])


def _pad_rows(a, rows=NP):
    return jnp.pad(a, ((0, rows - a.shape[0]),) + ((0, 0),) * (a.ndim - 1))


def _mk_table(g):
    z = jnp.zeros((8, 128), jnp.float32)
    return jnp.concatenate([g[:N, :128], z, g[:N, 128:], z], axis=0)


def _conv(tab, row_s0, row_s1, colrel_s, bstart, nch16):
    out = _sc_conv(tab, row_s0, row_s1, colrel_s, bstart, nch16)
    o = out.reshape(2, N, 128)
    return jnp.concatenate([o[0], o[1]], axis=1)


def kernel(x, params, edge_index, batch):
    p = params
    row = edge_index[0]
    col = edge_index[1]

    b = col // BASE
    cnt16 = jnp.bincount(b, length=NB)
    cap = ((cnt16 + C - 1) // C) * C
    pstart = jnp.concatenate([jnp.zeros((1,), jnp.int32),
                              jnp.cumsum(cap)[:-1].astype(jnp.int32)])
    nch16 = (cap // C).astype(jnp.int32)
    oh16 = (b[:, None] == jnp.arange(NB, dtype=b.dtype))
    rank = jnp.take_along_axis(jnp.cumsum(oh16.astype(jnp.int32), axis=0),
                               b[:, None].astype(jnp.int32), axis=1)[:, 0] - 1
    pos = pstart[b] + rank
    row_s = jnp.full((EP,), DUMMY, jnp.int32).at[pos].set(row)
    colrel_s = jnp.zeros((EP,), jnp.int32).at[pos].set(col - b * BASE)
    bstart = pstart.astype(jnp.int32)
    row_s1off = row_s + (N + 8)

    xp = _pad_rows(x)
    batp = jnp.concatenate([batch, jnp.full((NP - N,), B, jnp.int32)])

    onestab = jnp.concatenate([jnp.ones((N,), jnp.float32),
                               jnp.zeros((16,), jnp.float32)])
    dcnt1 = _sc_deg(onestab, row_s, colrel_s, bstart, nch16)
    deg1 = dcnt1[:, :BASE].reshape(N) + 1.0
    deg1p = _pad_rows(deg1[:, None], NP)[:, 0] + jnp.where(
        jnp.arange(NP) < N, 0.0, 1.0)

    g1 = _mm(xp, p['b1_c1_W'], jnp.zeros((F,), jnp.float32), deg=deg1p)
    s1 = _conv(_mk_table(g1), row_s, row_s1off, colrel_s, bstart, nch16)
    x1 = _gcn_post(_pad_rows(s1), g1, deg1p, p['b1_c1_b'])
    g2 = _mm(x1, p['b1_c2_W'], jnp.zeros((F,), jnp.float32), deg=deg1p)
    s2 = _conv(_mk_table(g2), row_s, row_s1off, colrel_s, bstart, nch16)
    x2 = _gcn_post(_pad_rows(s2), g2, deg1p, p['b1_c2_b'])
    xb1 = _mm(jnp.concatenate([x1, x2], axis=1), p['b1_lin_W'], p['b1_lin_b'],
              act="relu")

    wp = jnp.zeros((F, 128), jnp.float32).at[:, 0].set(p['pool_p'])
    sum1, max1, cnt, sc2d = _pools(xb1, batp, wp)
    score80 = sc2d[:, 0].reshape(NP // 128, 128)
    b80 = batp.reshape(NP // 128, 128)

    ks80, kf80, b2_80 = _topk(score80, b80, cnt)
    kscore = ks80.reshape(NP)
    keepf = kf80.reshape(NP)
    bat2 = b2_80.reshape(NP)

    keeptab = jnp.concatenate([keepf[:N], jnp.zeros((16,), jnp.float32)])
    dcnt = _sc_deg(keeptab, row_s, colrel_s, bstart, nch16)
    deg2 = dcnt[:, :BASE].reshape(N) + 1.0
    deg2p = _pad_rows(deg2[:, None], NP)[:, 0] + jnp.where(
        jnp.arange(NP) < N, 0.0, 1.0)

    g3 = _mm(xb1, p['b2_c1_W'], jnp.zeros((F,), jnp.float32),
             pre=kscore, deg=deg2p)
    s3 = _conv(_mk_table(g3), row_s, row_s1off, colrel_s, bstart, nch16)
    x1b = _gcn_post(_pad_rows(s3), g3, deg2p, p['b2_c1_b'])
    g4 = _mm(x1b, p['b2_c2_W'], jnp.zeros((F,), jnp.float32),
             pre=keepf, deg=deg2p)
    s4 = _conv(_mk_table(g4), row_s, row_s1off, colrel_s, bstart, nch16)
    x2b = _gcn_post(_pad_rows(s4), g4, deg2p, p['b2_c2_b'])
    xb2 = _mm(jnp.concatenate([x1b, x2b], axis=1), p['b2_lin_W'], p['b2_lin_b'],
              act="relu")

    sum2, max2, _, _ = _pools(xb2, bat2, wp)

    h = jnp.concatenate([sum1, max1, sum2, max2], axis=1)
    h = _mm(h, p['lin1_W'], p['lin1_b'], act="relu", bm=B)
    w2p = jnp.pad(p['lin2_W'], ((0, 0), (0, 118)))
    b2p = jnp.pad(p['lin2_b'], (0, 118))
    out = _mm(h, w2p, b2p, act="softmax", bm=B)
    return out[:, :10]


# speedup vs baseline: 4.5841x; 1.1036x over previous
import functools

import jax
import jax.numpy as jnp
import numpy as np
from jax import lax
from jax.experimental import pallas as pl
from jax.experimental.pallas import tpu as pltpu
from jax.experimental.pallas import tpu_sc as plsc

N = 10000
NP = 10240
F = 256
B = 64
NB = 16
BASE = N // NB
C = 128
E = 320000
EP = E + NB * C
TB = 20016
DUMMY = 10000
SLAB = BASE * 128

_mesh = plsc.VectorSubcoreMesh(core_axis_name="c", subcore_axis_name="s")
_scparams = pltpu.CompilerParams(needs_layout_passes=False)


def _mm_body(x_ref, w_ref, b_ref, pre_ref, deg_ref, o_ref, *, act, use_pre, use_deg):
    x = x_ref[...]
    if use_pre:
        x = x * pre_ref[...]
    acc = jnp.dot(x, w_ref[...], preferred_element_type=jnp.float32)
    acc = acc + b_ref[...]
    if act == "relu":
        acc = jnp.maximum(acc, 0.0)
    elif act == "softmax":
        col = lax.broadcasted_iota(jnp.int32, acc.shape, 1)
        acc = jnp.where(col < 10, acc, -1e30)
        acc = acc - jnp.max(acc, axis=1, keepdims=True)
        e = jnp.exp(acc)
        acc = e / jnp.sum(e, axis=1, keepdims=True)
    if use_deg:
        acc = acc * lax.rsqrt(jnp.maximum(deg_ref[...], 1e-12))
    o_ref[...] = acc


def _mm(x, w, b, act="none", pre=None, deg=None, bm=512):
    m, k = x.shape
    n = w.shape[1]
    use_pre = pre is not None
    use_deg = deg is not None
    if pre is None:
        pre = jnp.zeros((m, 1), jnp.float32)
    if deg is None:
        deg = jnp.ones((m, 1), jnp.float32)
    out = pl.pallas_call(
        functools.partial(_mm_body, act=act, use_pre=use_pre, use_deg=use_deg),
        grid=(m // bm,),
        in_specs=[
            pl.BlockSpec((bm, k), lambda i: (i, 0)),
            pl.BlockSpec((k, n), lambda i: (0, 0)),
            pl.BlockSpec((1, n), lambda i: (0, 0)),
            pl.BlockSpec((bm, 1), lambda i: (i, 0)),
            pl.BlockSpec((bm, 1), lambda i: (i, 0)),
        ],
        out_specs=pl.BlockSpec((bm, n), lambda i: (i, 0)),
        out_shape=jax.ShapeDtypeStruct((m, n), jnp.float32),
    )(x, w, b.reshape(1, n), pre.reshape(m, 1), deg.reshape(m, 1))
    return out


def _post_body(s_ref, g_ref, deg_ref, b_ref, o_ref):
    dinv = lax.rsqrt(jnp.maximum(deg_ref[...], 1e-12))
    o_ref[...] = jnp.maximum(dinv * (s_ref[...] + g_ref[...]) + b_ref[...], 0.0)


def _gcn_post(s, g, deg, b, bm=512):
    m = s.shape[0]
    return pl.pallas_call(
        _post_body,
        grid=(m // bm,),
        in_specs=[
            pl.BlockSpec((bm, F), lambda i: (i, 0)),
            pl.BlockSpec((bm, F), lambda i: (i, 0)),
            pl.BlockSpec((bm, 1), lambda i: (i, 0)),
            pl.BlockSpec((1, F), lambda i: (0, 0)),
        ],
        out_specs=pl.BlockSpec((bm, F), lambda i: (i, 0)),
        out_shape=jax.ShapeDtypeStruct((m, F), jnp.float32),
    )(s, g, deg.reshape(m, 1), b.reshape(1, F))


def _pool_body(x_ref, bat_ref, w_ref, sum_ref, max_ref, cnt_ref, sc_ref):
    i = pl.program_id(0)

    @pl.when(i == 0)
    def _():
        sum_ref[...] = jnp.zeros_like(sum_ref)
        max_ref[...] = jnp.full_like(max_ref, -jnp.inf)
        cnt_ref[...] = jnp.zeros_like(cnt_ref)

    x = x_ref[...]
    bat = bat_ref[...]
    wp = w_ref[...]
    nrm = jnp.sqrt(jnp.sum(wp * wp))
    sc_ref[...] = jnp.tanh(jnp.dot(x, wp, preferred_element_type=jnp.float32) / nrm)

    gi = lax.broadcasted_iota(jnp.int32, (x.shape[0], B), 1)
    oh = (bat == gi).astype(jnp.float32)
    sum_ref[...] += lax.dot_general(oh, x, (((0,), (0,)), ((), ())),
                                   preferred_element_type=jnp.float32)
    cnt_ref[...] += jnp.sum(oh, axis=0)[:, None]

    def body(g, _):
        mask = bat == g
        mx = jnp.max(jnp.where(mask, x, -jnp.inf), axis=0, keepdims=True)
        max_ref[pl.ds(g, 1), :] = jnp.maximum(max_ref[pl.ds(g, 1), :], mx)
        return 0
    lax.fori_loop(0, B, body, 0)

    @pl.when(i == pl.num_programs(0) - 1)
    def _():
        m = max_ref[...]
        max_ref[...] = jnp.where(jnp.isfinite(m), m, 0.0)


def _pools(x, bat, wp, bm=512):
    m = x.shape[0]
    return pl.pallas_call(
        _pool_body,
        grid=(m // bm,),
        in_specs=[
            pl.BlockSpec((bm, F), lambda i: (i, 0)),
            pl.BlockSpec((bm, 1), lambda i: (i, 0)),
            pl.BlockSpec((F, 128), lambda i: (0, 0)),
        ],
        out_specs=[
            pl.BlockSpec((B, F), lambda i: (0, 0)),
            pl.BlockSpec((B, F), lambda i: (0, 0)),
            pl.BlockSpec((B, 128), lambda i: (0, 0)),
            pl.BlockSpec((bm, 128), lambda i: (i, 0)),
        ],
        out_shape=[
            jax.ShapeDtypeStruct((B, F), jnp.float32),
            jax.ShapeDtypeStruct((B, F), jnp.float32),
            jax.ShapeDtypeStruct((B, 128), jnp.float32),
            jax.ShapeDtypeStruct((m, 128), jnp.float32),
        ],
    )(x, bat.reshape(m, 1), wp)


def _topk_body(s_ref, bat_ref, cnt_ref, ks_ref, kf_ref, b2_ref, t_s, t2_s, k_s, m_s):
    score = s_ref[...]
    bat = bat_ref[...]
    batf = bat.astype(jnp.float32)
    val = -(batf * 4.0 - score)
    u = lax.bitcast_convert_type(val, jnp.int32)
    sk = jnp.where(u < 0, u ^ np.int32(0x7FFFFFFF), u)
    msb = np.int32(-(2 ** 31))
    ss = sk ^ msb
    valid = bat < B

    def kinit(g, _):
        c = cnt_ref[pl.ds(g, 1), pl.ds(0, 1)][0, 0].astype(jnp.int32)
        k_s[g] = (4 * c + 4) // 5
        t_s[g] = 0
        t2_s[g] = 0
        return 0
    lax.fori_loop(0, B, kinit, 0)

    bits = [np.int32(-(2 ** 31))] + [np.int32(1 << b) for b in range(30, -1, -1)]
    for bit in bits:
        def gbody(g, _):
            cand = t_s[g] | bit
            scand = cand ^ msb
            ind = (ss >= scand) & (bat == g)
            cnt = jnp.sum(ind.astype(jnp.float32))
            t_s[g] = jnp.where(cnt >= k_s[g].astype(jnp.float32), cand, t_s[g])
            return 0
        lax.fori_loop(0, B, gbody, 0)

    def thr_body(g, acc):
        return jnp.where(bat == g, t_s[g], acc)
    thr = lax.fori_loop(0, B, thr_body, jnp.zeros_like(ss))
    sthr = thr ^ msb
    gt = ss > sthr
    tie = (ss == sthr) & valid

    def mb(g, _):
        ngt = jnp.sum((gt & (bat == g)).astype(jnp.float32))
        m_s[g] = k_s[g] - ngt.astype(jnp.int32)
        return 0
    lax.fori_loop(0, B, mb, 0)

    ridx = np.int32(16383) - (lax.broadcasted_iota(jnp.int32, score.shape, 0) * 128
                              + lax.broadcasted_iota(jnp.int32, score.shape, 1))
    for bit in [np.int32(1 << b) for b in range(13, -1, -1)]:
        def g2body(g, _):
            cand = t2_s[g] | bit
            ind = tie & (ridx >= cand) & (bat == g)
            cnt = jnp.sum(ind.astype(jnp.float32))
            t2_s[g] = jnp.where(cnt >= m_s[g].astype(jnp.float32), cand, t2_s[g])
            return 0
        lax.fori_loop(0, B, g2body, 0)

    def thr2_body(g, acc):
        return jnp.where(bat == g, t2_s[g], acc)
    thr2 = lax.fori_loop(0, B, thr2_body, jnp.zeros_like(ss))
    keep = (gt | (tie & (ridx >= thr2))) & valid
    keepf = keep.astype(jnp.float32)
    kf_ref[...] = keepf
    ks_ref[...] = keepf * score
    b2_ref[...] = jnp.where(keep, bat, B)


def _topk(s80, b80, cnt):
    return pl.pallas_call(
        _topk_body,
        grid=(1,),
        in_specs=[
            pl.BlockSpec((NP // 128, 128), lambda i: (0, 0)),
            pl.BlockSpec((NP // 128, 128), lambda i: (0, 0)),
            pl.BlockSpec((B, 128), lambda i: (0, 0)),
        ],
        out_specs=[
            pl.BlockSpec((NP // 128, 128), lambda i: (0, 0)),
            pl.BlockSpec((NP // 128, 128), lambda i: (0, 0)),
            pl.BlockSpec((NP // 128, 128), lambda i: (0, 0)),
        ],
        out_shape=[
            jax.ShapeDtypeStruct((NP // 128, 128), jnp.float32),
            jax.ShapeDtypeStruct((NP // 128, 128), jnp.float32),
            jax.ShapeDtypeStruct((NP // 128, 128), jnp.int32),
        ],
        scratch_shapes=[pltpu.SMEM((B,), jnp.int32)] * 4,
    )(s80, b80, cnt)


@functools.partial(
    pl.kernel, mesh=_mesh, compiler_params=_scparams,
    out_type=jax.ShapeDtypeStruct((32, SLAB), jnp.float32),
    scratch_types=[
        pltpu.VMEM((16,), jnp.int32),
        pltpu.VMEM((16,), jnp.int32),
        pltpu.VMEM((C,), jnp.int32),
        pltpu.VMEM((C,), jnp.int32),
        pltpu.VMEM((C,), jnp.int32),
        pltpu.VMEM((C,), jnp.int32),
        pltpu.VMEM((C, 128), jnp.float32),
        pltpu.VMEM((C, 128), jnp.float32),
        pltpu.VMEM((SLAB,), jnp.float32),
        pltpu.SemaphoreType.DMA,
        pltpu.SemaphoreType.DMA,
    ],
)
def _sc_conv(tab_hbm, row0_hbm, row1_hbm, col_hbm, bs_hbm, nc_hbm, out_hbm,
             bsv, ncv, rowvA, colvA, rowvB, colvB, bufA, bufB, slab,
             semA, semB):
    tid = lax.axis_index("s")
    cid = lax.axis_index("c")
    IOTA = lax.iota(jnp.int32, 16)

    pltpu.sync_copy(bs_hbm, bsv)
    pltpu.sync_copy(nc_hbm, ncv)
    lo = jnp.sum(jnp.where(IOTA == tid, bsv[...], 0))
    nch = jnp.sum(jnp.where(IOTA == tid, ncv[...], 0))

    zero = jnp.zeros((16,), jnp.float32)

    def zbody(i, _):
        for j in range(8):
            slab[pl.ds(i * 128 + j * 16, 16)] = zero
        return 0
    lax.fori_loop(0, SLAB // 128, zbody, 0)

    def start(j, rv, cv, buf, sem):
        base = pl.multiple_of(lo + j * C, 128)

        @pl.when(cid == 0)
        def _():
            pltpu.sync_copy(row0_hbm.at[pl.ds(base, C)], rv)

        @pl.when(cid == 1)
        def _():
            pltpu.sync_copy(row1_hbm.at[pl.ds(base, C)], rv)
        pltpu.sync_copy(col_hbm.at[pl.ds(base, C)], cv)
        pltpu.async_copy(tab_hbm.at[rv], buf, sem)

    def process(rv, cv, buf, sem):
        pltpu.make_async_copy(tab_hbm.at[rv], buf, sem).wait()

        def ebody(e, _):
            for u in range(2):
                ei = e * 2 + u
                cvec = plsc.load_gather(cv, [jnp.full((16,), ei, jnp.int32)])
                a0 = cvec * 128 + IOTA
                for kk in range(8):
                    dat = buf[ei, pl.ds(kk * 16, 16)]
                    plsc.addupdate_scatter(slab, [a0 + (kk * 16)], dat)
            return 0
        lax.fori_loop(0, C // 2, ebody, 0)

    @pl.when(nch > 0)
    def _():
        start(0, rowvA, colvA, bufA, semA)

    def body(i, _):
        c1 = 2 * i + 1

        @pl.when(c1 < nch)
        def _():
            start(c1, rowvB, colvB, bufB, semB)
        process(rowvA, colvA, bufA, semA)

        @pl.when(c1 < nch)
        def _():
            @pl.when(c1 + 1 < nch)
            def _():
                start(c1 + 1, rowvA, colvA, bufA, semA)
            process(rowvB, colvB, bufB, semB)
        return 0

    lax.fori_loop(0, (nch + 1) // 2, body, 0)
    pltpu.sync_copy(slab, out_hbm.at[cid * 16 + tid])


@functools.partial(
    pl.kernel, mesh=_mesh, compiler_params=_scparams,
    out_type=jax.ShapeDtypeStruct((16, 640), jnp.float32),
    scratch_types=[
        pltpu.VMEM((16,), jnp.int32),
        pltpu.VMEM((16,), jnp.int32),
        pltpu.VMEM((10016,), jnp.float32),
        pltpu.VMEM((C,), jnp.int32),
        pltpu.VMEM((C,), jnp.int32),
        pltpu.VMEM((640,), jnp.float32),
    ],
)
def _sc_deg(keep_hbm, row_hbm, col_hbm, bs_hbm, nc_hbm, out_hbm,
            bsv, ncv, keepv, rowv, colv, slab):
    tid = lax.axis_index("s")
    cid = lax.axis_index("c")
    IOTA = lax.iota(jnp.int32, 16)

    @pl.when(cid == 0)
    def _():
        pltpu.sync_copy(bs_hbm, bsv)
        pltpu.sync_copy(nc_hbm, ncv)
        pltpu.sync_copy(keep_hbm, keepv)
        lo = jnp.sum(jnp.where(IOTA == tid, bsv[...], 0))
        nch = jnp.sum(jnp.where(IOTA == tid, ncv[...], 0))
        zero = jnp.zeros((16,), jnp.float32)
        for i in range(640 // 16):
            slab[pl.ds(i * 16, 16)] = zero

        def body(i, _):
            base = pl.multiple_of(lo + i * C, 128)
            pltpu.sync_copy(row_hbm.at[pl.ds(base, C)], rowv)
            pltpu.sync_copy(col_hbm.at[pl.ds(base, C)], colv)

            def ebody(j, _):
                for u in range(2):
                    jj = (j * 2 + u) * 16
                    kv = plsc.load_gather(keepv, [rowv[pl.ds(jj, 16)]])
                    plsc.addupdate_scatter(slab, [colv[pl.ds(jj, 16)]], kv)
                return 0
            lax.fori_loop(0, C // 32, ebody, 0)
            return 0

        lax.fori_loop(0, nch, body, 0)
        pltpu.sync_copy(slab, out_hbm.at[tid])


def _pad_rows(a, rows=NP):
    return jnp.pad(a, ((0, rows - a.shape[0]),) + ((0, 0),) * (a.ndim - 1))


def _mk_table(g):
    z = jnp.zeros((8, 128), jnp.float32)
    return jnp.concatenate([g[:N, :128], z, g[:N, 128:], z], axis=0)


def _conv(tab, row_s0, row_s1, colrel_s, bstart, nch16):
    out = _sc_conv(tab, row_s0, row_s1, colrel_s, bstart, nch16)
    o = out.reshape(2, N, 128)
    return jnp.concatenate([o[0], o[1]], axis=1)


def kernel(x, params, edge_index, batch):
    p = params
    row = edge_index[0]
    col = edge_index[1]

    b = col // BASE
    cnt16 = jnp.bincount(b, length=NB)
    cap = ((cnt16 + C - 1) // C) * C
    pstart = jnp.concatenate([jnp.zeros((1,), jnp.int32),
                              jnp.cumsum(cap)[:-1].astype(jnp.int32)])
    nch16 = (cap // C).astype(jnp.int32)
    oh16 = (b[:, None] == jnp.arange(NB, dtype=b.dtype))
    rank = jnp.take_along_axis(jnp.cumsum(oh16.astype(jnp.int32), axis=0),
                               b[:, None].astype(jnp.int32), axis=1)[:, 0] - 1
    pos = pstart[b] + rank
    row_s = jnp.full((EP,), DUMMY, jnp.int32).at[pos].set(row)
    colrel_s = jnp.zeros((EP,), jnp.int32).at[pos].set(col - b * BASE)
    bstart = pstart.astype(jnp.int32)
    row_s1off = row_s + (N + 8)

    xp = _pad_rows(x)
    batp = jnp.concatenate([batch, jnp.full((NP - N,), B, jnp.int32)])

    onestab = jnp.concatenate([jnp.ones((N,), jnp.float32),
                               jnp.zeros((16,), jnp.float32)])
    dcnt1 = _sc_deg(onestab, row_s, colrel_s, bstart, nch16)
    deg1 = dcnt1[:, :BASE].reshape(N) + 1.0
    deg1p = _pad_rows(deg1[:, None], NP)[:, 0] + jnp.where(
        jnp.arange(NP) < N, 0.0, 1.0)

    g1 = _mm(xp, p['b1_c1_W'], jnp.zeros((F,), jnp.float32), deg=deg1p)
    s1 = _conv(_mk_table(g1), row_s, row_s1off, colrel_s, bstart, nch16)
    x1 = _gcn_post(_pad_rows(s1), g1, deg1p, p['b1_c1_b'])
    g2 = _mm(x1, p['b1_c2_W'], jnp.zeros((F,), jnp.float32), deg=deg1p)
    s2 = _conv(_mk_table(g2), row_s, row_s1off, colrel_s, bstart, nch16)
    x2 = _gcn_post(_pad_rows(s2), g2, deg1p, p['b1_c2_b'])
    xb1 = _mm(jnp.concatenate([x1, x2], axis=1), p['b1_lin_W'], p['b1_lin_b'],
              act="relu")

    wp = jnp.zeros((F, 128), jnp.float32).at[:, 0].set(p['pool_p'])
    sum1, max1, cnt, sc2d = _pools(xb1, batp, wp)
    score80 = sc2d[:, 0].reshape(NP // 128, 128)
    b80 = batp.reshape(NP // 128, 128)

    ks80, kf80, b2_80 = _topk(score80, b80, cnt)
    kscore = ks80.reshape(NP)
    keepf = kf80.reshape(NP)
    bat2 = b2_80.reshape(NP)

    keeptab = jnp.concatenate([keepf[:N], jnp.zeros((16,), jnp.float32)])
    dcnt = _sc_deg(keeptab, row_s, colrel_s, bstart, nch16)
    deg2 = dcnt[:, :BASE].reshape(N) + 1.0
    deg2p = _pad_rows(deg2[:, None], NP)[:, 0] + jnp.where(
        jnp.arange(NP) < N, 0.0, 1.0)

    g3 = _mm(xb1, p['b2_c1_W'], jnp.zeros((F,), jnp.float32),
             pre=kscore, deg=deg2p)
    s3 = _conv(_mk_table(g3), row_s, row_s1off, colrel_s, bstart, nch16)
    x1b = _gcn_post(_pad_rows(s3), g3, deg2p, p['b2_c1_b'])
    g4 = _mm(x1b, p['b2_c2_W'], jnp.zeros((F,), jnp.float32),
             pre=keepf, deg=deg2p)
    s4 = _conv(_mk_table(g4), row_s, row_s1off, colrel_s, bstart, nch16)
    x2b = _gcn_post(_pad_rows(s4), g4, deg2p, p['b2_c2_b'])
    xb2 = _mm(jnp.concatenate([x1b, x2b], axis=1), p['b2_lin_W'], p['b2_lin_b'],
              act="relu")

    sum2, max2, _, _ = _pools(xb2, bat2, wp)

    h = jnp.concatenate([sum1, max1, sum2, max2], axis=1)
    h = _mm(h, p['lin1_W'], p['lin1_b'], act="relu", bm=B)
    w2p = jnp.pad(p['lin2_W'], ((0, 0), (0, 118)))
    b2p = jnp.pad(p['lin2_b'], (0, 118))
    out = _mm(h, w2p, b2p, act="softmax", bm=B)
    return out[:, :10]


# speedup vs baseline: 4.7295x; 1.0317x over previous
import functools

import jax
import jax.numpy as jnp
import numpy as np
from jax import lax
from jax.experimental import pallas as pl
from jax.experimental.pallas import tpu as pltpu
from jax.experimental.pallas import tpu_sc as plsc

N = 10000
NP = 10240
F = 256
B = 64
NB = 32
BASE = 313
C = 64
E = 320000
EP = E + NB * C
DUMMY = 10000
SLAB = BASE * F

_mesh = plsc.VectorSubcoreMesh(core_axis_name="c", subcore_axis_name="s")
_scparams = pltpu.CompilerParams(needs_layout_passes=False)


def _mm_body(x_ref, w_ref, b_ref, pre_ref, deg_ref, o_ref, *, act, use_pre, use_deg):
    x = x_ref[...]
    if use_pre:
        x = x * pre_ref[...]
    acc = jnp.dot(x, w_ref[...], preferred_element_type=jnp.float32)
    acc = acc + b_ref[...]
    if act == "relu":
        acc = jnp.maximum(acc, 0.0)
    elif act == "softmax":
        col = lax.broadcasted_iota(jnp.int32, acc.shape, 1)
        acc = jnp.where(col < 10, acc, -1e30)
        acc = acc - jnp.max(acc, axis=1, keepdims=True)
        e = jnp.exp(acc)
        acc = e / jnp.sum(e, axis=1, keepdims=True)
    if use_deg:
        acc = acc * lax.rsqrt(jnp.maximum(deg_ref[...], 1e-12))
    o_ref[...] = acc


def _mm(x, w, b, act="none", pre=None, deg=None, bm=512):
    m, k = x.shape
    n = w.shape[1]
    use_pre = pre is not None
    use_deg = deg is not None
    if pre is None:
        pre = jnp.zeros((m, 1), jnp.float32)
    if deg is None:
        deg = jnp.ones((m, 1), jnp.float32)
    out = pl.pallas_call(
        functools.partial(_mm_body, act=act, use_pre=use_pre, use_deg=use_deg),
        grid=(m // bm,),
        in_specs=[
            pl.BlockSpec((bm, k), lambda i: (i, 0)),
            pl.BlockSpec((k, n), lambda i: (0, 0)),
            pl.BlockSpec((1, n), lambda i: (0, 0)),
            pl.BlockSpec((bm, 1), lambda i: (i, 0)),
            pl.BlockSpec((bm, 1), lambda i: (i, 0)),
        ],
        out_specs=pl.BlockSpec((bm, n), lambda i: (i, 0)),
        out_shape=jax.ShapeDtypeStruct((m, n), jnp.float32),
    )(x, w, b.reshape(1, n), pre.reshape(m, 1), deg.reshape(m, 1))
    return out


def _post_body(s_ref, g_ref, deg_ref, b_ref, o_ref):
    dinv = lax.rsqrt(jnp.maximum(deg_ref[...], 1e-12))
    o_ref[...] = jnp.maximum(dinv * (s_ref[...] + g_ref[...]) + b_ref[...], 0.0)


def _gcn_post(s, g, deg, b, bm=512):
    m = s.shape[0]
    return pl.pallas_call(
        _post_body,
        grid=(m // bm,),
        in_specs=[
            pl.BlockSpec((bm, F), lambda i: (i, 0)),
            pl.BlockSpec((bm, F), lambda i: (i, 0)),
            pl.BlockSpec((bm, 1), lambda i: (i, 0)),
            pl.BlockSpec((1, F), lambda i: (0, 0)),
        ],
        out_specs=pl.BlockSpec((bm, F), lambda i: (i, 0)),
        out_shape=jax.ShapeDtypeStruct((m, F), jnp.float32),
    )(s, g, deg.reshape(m, 1), b.reshape(1, F))


def _pool_body(x_ref, bat_ref, w_ref, sum_ref, max_ref, cnt_ref, sc_ref):
    i = pl.program_id(0)

    @pl.when(i == 0)
    def _():
        sum_ref[...] = jnp.zeros_like(sum_ref)
        max_ref[...] = jnp.full_like(max_ref, -jnp.inf)
        cnt_ref[...] = jnp.zeros_like(cnt_ref)

    x = x_ref[...]
    bat = bat_ref[...]
    wp = w_ref[...]
    nrm = jnp.sqrt(jnp.sum(wp * wp))
    sc_ref[...] = jnp.tanh(jnp.dot(x, wp, preferred_element_type=jnp.float32) / nrm)

    gi = lax.broadcasted_iota(jnp.int32, (x.shape[0], B), 1)
    oh = (bat == gi).astype(jnp.float32)
    sum_ref[...] += lax.dot_general(oh, x, (((0,), (0,)), ((), ())),
                                   preferred_element_type=jnp.float32)
    cnt_ref[...] += jnp.sum(oh, axis=0)[:, None]

    def body(g, _):
        mask = bat == g
        mx = jnp.max(jnp.where(mask, x, -jnp.inf), axis=0, keepdims=True)
        max_ref[pl.ds(g, 1), :] = jnp.maximum(max_ref[pl.ds(g, 1), :], mx)
        return 0
    lax.fori_loop(0, B, body, 0)

    @pl.when(i == pl.num_programs(0) - 1)
    def _():
        m = max_ref[...]
        max_ref[...] = jnp.where(jnp.isfinite(m), m, 0.0)


def _pools(x, bat, wp, bm=512):
    m = x.shape[0]
    return pl.pallas_call(
        _pool_body,
        grid=(m // bm,),
        in_specs=[
            pl.BlockSpec((bm, F), lambda i: (i, 0)),
            pl.BlockSpec((bm, 1), lambda i: (i, 0)),
            pl.BlockSpec((F, 128), lambda i: (0, 0)),
        ],
        out_specs=[
            pl.BlockSpec((B, F), lambda i: (0, 0)),
            pl.BlockSpec((B, F), lambda i: (0, 0)),
            pl.BlockSpec((B, 128), lambda i: (0, 0)),
            pl.BlockSpec((bm, 128), lambda i: (i, 0)),
        ],
        out_shape=[
            jax.ShapeDtypeStruct((B, F), jnp.float32),
            jax.ShapeDtypeStruct((B, F), jnp.float32),
            jax.ShapeDtypeStruct((B, 128), jnp.float32),
            jax.ShapeDtypeStruct((m, 128), jnp.float32),
        ],
    )(x, bat.reshape(m, 1), wp)


def _topk_body(s_ref, bat_ref, cnt_ref, ks_ref, kf_ref, b2_ref, t_s, t2_s, k_s, m_s):
    score = s_ref[...]
    bat = bat_ref[...]
    batf = bat.astype(jnp.float32)
    val = -(batf * 4.0 - score)
    u = lax.bitcast_convert_type(val, jnp.int32)
    sk = jnp.where(u < 0, u ^ np.int32(0x7FFFFFFF), u)
    msb = np.int32(-(2 ** 31))
    ss = sk ^ msb
    valid = bat < B

    def kinit(g, _):
        c = cnt_ref[pl.ds(g, 1), pl.ds(0, 1)][0, 0].astype(jnp.int32)
        k_s[g] = (4 * c + 4) // 5
        t_s[g] = 0
        t2_s[g] = 0
        return 0
    lax.fori_loop(0, B, kinit, 0)

    bits = [np.int32(-(2 ** 31))] + [np.int32(1 << b) for b in range(30, -1, -1)]
    for bit in bits:
        def gbody(g, _):
            cand = t_s[g] | bit
            scand = cand ^ msb
            ind = (ss >= scand) & (bat == g)
            cnt = jnp.sum(ind.astype(jnp.float32))
            t_s[g] = jnp.where(cnt >= k_s[g].astype(jnp.float32), cand, t_s[g])
            return 0
        lax.fori_loop(0, B, gbody, 0)

    def thr_body(g, acc):
        return jnp.where(bat == g, t_s[g], acc)
    thr = lax.fori_loop(0, B, thr_body, jnp.zeros_like(ss))
    sthr = thr ^ msb
    gt = ss > sthr
    tie = (ss == sthr) & valid

    def mb(g, _):
        ngt = jnp.sum((gt & (bat == g)).astype(jnp.float32))
        m_s[g] = k_s[g] - ngt.astype(jnp.int32)
        return 0
    lax.fori_loop(0, B, mb, 0)

    ridx = np.int32(16383) - (lax.broadcasted_iota(jnp.int32, score.shape, 0) * 128
                              + lax.broadcasted_iota(jnp.int32, score.shape, 1))
    for bit in [np.int32(1 << b) for b in range(13, -1, -1)]:
        def g2body(g, _):
            cand = t2_s[g] | bit
            ind = tie & (ridx >= cand) & (bat == g)
            cnt = jnp.sum(ind.astype(jnp.float32))
            t2_s[g] = jnp.where(cnt >= m_s[g].astype(jnp.float32), cand, t2_s[g])
            return 0
        lax.fori_loop(0, B, g2body, 0)

    def thr2_body(g, acc):
        return jnp.where(bat == g, t2_s[g], acc)
    thr2 = lax.fori_loop(0, B, thr2_body, jnp.zeros_like(ss))
    keep = (gt | (tie & (ridx >= thr2))) & valid
    keepf = keep.astype(jnp.float32)
    kf_ref[...] = keepf
    ks_ref[...] = keepf * score
    b2_ref[...] = jnp.where(keep, bat, B)


def _topk(s80, b80, cnt):
    return pl.pallas_call(
        _topk_body,
        grid=(1,),
        in_specs=[
            pl.BlockSpec((NP // 128, 128), lambda i: (0, 0)),
            pl.BlockSpec((NP // 128, 128), lambda i: (0, 0)),
            pl.BlockSpec((B, 128), lambda i: (0, 0)),
        ],
        out_specs=[
            pl.BlockSpec((NP // 128, 128), lambda i: (0, 0)),
            pl.BlockSpec((NP // 128, 128), lambda i: (0, 0)),
            pl.BlockSpec((NP // 128, 128), lambda i: (0, 0)),
        ],
        out_shape=[
            jax.ShapeDtypeStruct((NP // 128, 128), jnp.float32),
            jax.ShapeDtypeStruct((NP // 128, 128), jnp.float32),
            jax.ShapeDtypeStruct((NP // 128, 128), jnp.int32),
        ],
        scratch_shapes=[pltpu.SMEM((B,), jnp.int32)] * 4,
    )(s80, b80, cnt)


@functools.partial(
    pl.kernel, mesh=_mesh, compiler_params=_scparams,
    out_type=jax.ShapeDtypeStruct((32, SLAB), jnp.float32),
    scratch_types=[
        pltpu.VMEM((16,), jnp.int32),
        pltpu.VMEM((16,), jnp.int32),
        pltpu.VMEM((C,), jnp.int32),
        pltpu.VMEM((C,), jnp.int32),
        pltpu.VMEM((C,), jnp.int32),
        pltpu.VMEM((C,), jnp.int32),
        pltpu.VMEM((C, F), jnp.float32),
        pltpu.VMEM((C, F), jnp.float32),
        pltpu.VMEM((SLAB,), jnp.float32),
        pltpu.SemaphoreType.DMA,
        pltpu.SemaphoreType.DMA,
    ],
)
def _sc_conv(tab_hbm, row_hbm, col_hbm, bs_hbm, nc_hbm, out_hbm,
             bsv, ncv, rowvA, colvA, rowvB, colvB, bufA, bufB, slab,
             semA, semB):
    tid = lax.axis_index("s")
    cid = lax.axis_index("c")
    IOTA = lax.iota(jnp.int32, 16)

    pltpu.sync_copy(bs_hbm.at[cid], bsv)
    pltpu.sync_copy(nc_hbm.at[cid], ncv)
    lo = jnp.sum(jnp.where(IOTA == tid, bsv[...], 0))
    nch = jnp.sum(jnp.where(IOTA == tid, ncv[...], 0))

    zero = jnp.zeros((16,), jnp.float32)

    def zbody(i, _):
        for j in range(8):
            slab[pl.ds(i * 128 + j * 16, 16)] = zero
        return 0
    lax.fori_loop(0, SLAB // 128, zbody, 0)

    def start(j, rv, cv, buf, sem):
        base = pl.multiple_of(lo + j * C, C)
        pltpu.sync_copy(row_hbm.at[pl.ds(base, C)], rv)
        pltpu.sync_copy(col_hbm.at[pl.ds(base, C)], cv)
        pltpu.async_copy(tab_hbm.at[rv], buf, sem)

    def process(rv, cv, buf, sem):
        pltpu.make_async_copy(tab_hbm.at[rv], buf, sem).wait()

        def ebody(e, _):
            for u in range(2):
                ei = e * 2 + u
                cvec = plsc.load_gather(cv, [jnp.full((16,), ei, jnp.int32)])
                a0 = cvec * F + IOTA
                for kk in range(16):
                    dat = buf[ei, pl.ds(kk * 16, 16)]
                    plsc.addupdate_scatter(slab, [a0 + (kk * 16)], dat)
            return 0
        lax.fori_loop(0, C // 2, ebody, 0)

    @pl.when(nch > 0)
    def _():
        start(0, rowvA, colvA, bufA, semA)

    def body(i, _):
        c1 = 2 * i + 1

        @pl.when(c1 < nch)
        def _():
            start(c1, rowvB, colvB, bufB, semB)
        process(rowvA, colvA, bufA, semA)

        @pl.when(c1 < nch)
        def _():
            @pl.when(c1 + 1 < nch)
            def _():
                start(c1 + 1, rowvA, colvA, bufA, semA)
            process(rowvB, colvB, bufB, semB)
        return 0

    lax.fori_loop(0, (nch + 1) // 2, body, 0)
    pltpu.sync_copy(slab, out_hbm.at[cid * 16 + tid])


@functools.partial(
    pl.kernel, mesh=_mesh, compiler_params=_scparams,
    out_type=jax.ShapeDtypeStruct((32, 320), jnp.float32),
    scratch_types=[
        pltpu.VMEM((16,), jnp.int32),
        pltpu.VMEM((16,), jnp.int32),
        pltpu.VMEM((10016,), jnp.float32),
        pltpu.VMEM((C,), jnp.int32),
        pltpu.VMEM((C,), jnp.int32),
        pltpu.VMEM((320,), jnp.float32),
    ],
)
def _sc_deg(keep_hbm, row_hbm, col_hbm, bs_hbm, nc_hbm, out_hbm,
            bsv, ncv, keepv, rowv, colv, slab):
    tid = lax.axis_index("s")
    cid = lax.axis_index("c")
    IOTA = lax.iota(jnp.int32, 16)

    pltpu.sync_copy(bs_hbm.at[cid], bsv)
    pltpu.sync_copy(nc_hbm.at[cid], ncv)
    pltpu.sync_copy(keep_hbm, keepv)
    lo = jnp.sum(jnp.where(IOTA == tid, bsv[...], 0))
    nch = jnp.sum(jnp.where(IOTA == tid, ncv[...], 0))
    zero = jnp.zeros((16,), jnp.float32)
    for i in range(320 // 16):
        slab[pl.ds(i * 16, 16)] = zero

    def body(i, _):
        base = pl.multiple_of(lo + i * C, C)
        pltpu.sync_copy(row_hbm.at[pl.ds(base, C)], rowv)
        pltpu.sync_copy(col_hbm.at[pl.ds(base, C)], colv)

        def ebody(j, _):
            for u in range(2):
                jj = (j * 2 + u) * 16
                kv = plsc.load_gather(keepv, [rowv[pl.ds(jj, 16)]])
                plsc.addupdate_scatter(slab, [colv[pl.ds(jj, 16)]], kv)
            return 0
        lax.fori_loop(0, C // 32, ebody, 0)
        return 0

    lax.fori_loop(0, nch, body, 0)
    pltpu.sync_copy(slab, out_hbm.at[cid * 16 + tid])


def _pad_rows(a, rows=NP):
    return jnp.pad(a, ((0, rows - a.shape[0]),) + ((0, 0),) * (a.ndim - 1))


def _mk_table(g):
    z = jnp.zeros((8, F), jnp.float32)
    return jnp.concatenate([g[:N], z], axis=0)


def _conv(tab, row_s, colrel_s, bstart, nch16):
    out = _sc_conv(tab, row_s, colrel_s, bstart, nch16)
    return out.reshape(NB * BASE, F)[:N]


def kernel(x, params, edge_index, batch):
    p = params
    row = edge_index[0]
    col = edge_index[1]

    b = col // BASE
    cnt16 = jnp.bincount(b, length=NB)
    cap = ((cnt16 + C - 1) // C) * C
    pstart = jnp.concatenate([jnp.zeros((1,), jnp.int32),
                              jnp.cumsum(cap)[:-1].astype(jnp.int32)])
    nch16 = (cap // C).astype(jnp.int32)
    oh16 = (b[:, None] == jnp.arange(NB, dtype=b.dtype))
    rank = jnp.take_along_axis(jnp.cumsum(oh16.astype(jnp.int32), axis=0),
                               b[:, None].astype(jnp.int32), axis=1)[:, 0] - 1
    pos = pstart[b] + rank
    row_s = jnp.full((EP,), DUMMY, jnp.int32).at[pos].set(row)
    colrel_s = jnp.zeros((EP,), jnp.int32).at[pos].set(col - b * BASE)
    bstart = pstart.astype(jnp.int32).reshape(2, 16)
    nch16 = nch16.reshape(2, 16)

    xp = _pad_rows(x)
    batp = jnp.concatenate([batch, jnp.full((NP - N,), B, jnp.int32)])

    onestab = jnp.concatenate([jnp.ones((N,), jnp.float32),
                               jnp.zeros((16,), jnp.float32)])
    dcnt1 = _sc_deg(onestab, row_s, colrel_s, bstart, nch16)
    deg1 = dcnt1[:, :BASE].reshape(NB * BASE)[:N] + 1.0
    deg1p = _pad_rows(deg1[:, None], NP)[:, 0] + jnp.where(
        jnp.arange(NP) < N, 0.0, 1.0)

    g1 = _mm(xp, p['b1_c1_W'], jnp.zeros((F,), jnp.float32), deg=deg1p)
    s1 = _conv(_mk_table(g1), row_s, colrel_s, bstart, nch16)
    x1 = _gcn_post(_pad_rows(s1), g1, deg1p, p['b1_c1_b'])
    g2 = _mm(x1, p['b1_c2_W'], jnp.zeros((F,), jnp.float32), deg=deg1p)
    s2 = _conv(_mk_table(g2), row_s, colrel_s, bstart, nch16)
    x2 = _gcn_post(_pad_rows(s2), g2, deg1p, p['b1_c2_b'])
    xb1 = _mm(jnp.concatenate([x1, x2], axis=1), p['b1_lin_W'], p['b1_lin_b'],
              act="relu")

    wp = jnp.zeros((F, 128), jnp.float32).at[:, 0].set(p['pool_p'])
    sum1, max1, cnt, sc2d = _pools(xb1, batp, wp)
    score80 = sc2d[:, 0].reshape(NP // 128, 128)
    b80 = batp.reshape(NP // 128, 128)

    ks80, kf80, b2_80 = _topk(score80, b80, cnt)
    kscore = ks80.reshape(NP)
    keepf = kf80.reshape(NP)
    bat2 = b2_80.reshape(NP)

    keeptab = jnp.concatenate([keepf[:N], jnp.zeros((16,), jnp.float32)])
    dcnt = _sc_deg(keeptab, row_s, colrel_s, bstart, nch16)
    deg2 = dcnt[:, :BASE].reshape(NB * BASE)[:N] + 1.0
    deg2p = _pad_rows(deg2[:, None], NP)[:, 0] + jnp.where(
        jnp.arange(NP) < N, 0.0, 1.0)

    g3 = _mm(xb1, p['b2_c1_W'], jnp.zeros((F,), jnp.float32),
             pre=kscore, deg=deg2p)
    s3 = _conv(_mk_table(g3), row_s, colrel_s, bstart, nch16)
    x1b = _gcn_post(_pad_rows(s3), g3, deg2p, p['b2_c1_b'])
    g4 = _mm(x1b, p['b2_c2_W'], jnp.zeros((F,), jnp.float32),
             pre=keepf, deg=deg2p)
    s4 = _conv(_mk_table(g4), row_s, colrel_s, bstart, nch16)
    x2b = _gcn_post(_pad_rows(s4), g4, deg2p, p['b2_c2_b'])
    xb2 = _mm(jnp.concatenate([x1b, x2b], axis=1), p['b2_lin_W'], p['b2_lin_b'],
              act="relu")

    sum2, max2, _, _ = _pools(xb2, bat2, wp)

    h = jnp.concatenate([sum1, max1, sum2, max2], axis=1)
    h = _mm(h, p['lin1_W'], p['lin1_b'], act="relu", bm=B)
    w2p = jnp.pad(p['lin2_W'], ((0, 0), (0, 118)))
    b2p = jnp.pad(p['lin2_b'], (0, 118))
    out = _mm(h, w2p, b2p, act="softmax", bm=B)
    return out[:, :10]


# speedup vs baseline: 4.9783x; 1.0526x over previous
import functools

import jax
import jax.numpy as jnp
import numpy as np
from jax import lax
from jax.experimental import pallas as pl
from jax.experimental.pallas import tpu as pltpu
from jax.experimental.pallas import tpu_sc as plsc

N = 10000
NP = 10240
F = 256
B = 64
NB = 32
BASE = 313
C = 64
E = 320000
EP = E + NB * C + 16 * C
DUMMY = 10000
SLAB = BASE * F

_mesh = plsc.VectorSubcoreMesh(core_axis_name="c", subcore_axis_name="s")
_scparams = pltpu.CompilerParams(needs_layout_passes=False)


def _mm_body(x_ref, w_ref, b_ref, pre_ref, deg_ref, o_ref, *, act, use_pre, use_deg):
    x = x_ref[...]
    if use_pre:
        x = x * pre_ref[...]
    acc = jnp.dot(x, w_ref[...], preferred_element_type=jnp.float32)
    acc = acc + b_ref[...]
    if act == "relu":
        acc = jnp.maximum(acc, 0.0)
    elif act == "softmax":
        col = lax.broadcasted_iota(jnp.int32, acc.shape, 1)
        acc = jnp.where(col < 10, acc, -1e30)
        acc = acc - jnp.max(acc, axis=1, keepdims=True)
        e = jnp.exp(acc)
        acc = e / jnp.sum(e, axis=1, keepdims=True)
    if use_deg:
        acc = acc * lax.rsqrt(jnp.maximum(deg_ref[...], 1e-12))
    o_ref[...] = acc


def _mm(x, w, b, act="none", pre=None, deg=None, bm=512):
    m, k = x.shape
    n = w.shape[1]
    use_pre = pre is not None
    use_deg = deg is not None
    if pre is None:
        pre = jnp.zeros((m, 1), jnp.float32)
    if deg is None:
        deg = jnp.ones((m, 1), jnp.float32)
    out = pl.pallas_call(
        functools.partial(_mm_body, act=act, use_pre=use_pre, use_deg=use_deg),
        grid=(m // bm,),
        in_specs=[
            pl.BlockSpec((bm, k), lambda i: (i, 0)),
            pl.BlockSpec((k, n), lambda i: (0, 0)),
            pl.BlockSpec((1, n), lambda i: (0, 0)),
            pl.BlockSpec((bm, 1), lambda i: (i, 0)),
            pl.BlockSpec((bm, 1), lambda i: (i, 0)),
        ],
        out_specs=pl.BlockSpec((bm, n), lambda i: (i, 0)),
        out_shape=jax.ShapeDtypeStruct((m, n), jnp.float32),
    )(x, w, b.reshape(1, n), pre.reshape(m, 1), deg.reshape(m, 1))
    return out


def _post_body(s_ref, g_ref, deg_ref, b_ref, o_ref):
    dinv = lax.rsqrt(jnp.maximum(deg_ref[...], 1e-12))
    o_ref[...] = jnp.maximum(dinv * (s_ref[...] + g_ref[...]) + b_ref[...], 0.0)


def _gcn_post(s, g, deg, b, bm=512):
    m = s.shape[0]
    return pl.pallas_call(
        _post_body,
        grid=(m // bm,),
        in_specs=[
            pl.BlockSpec((bm, F), lambda i: (i, 0)),
            pl.BlockSpec((bm, F), lambda i: (i, 0)),
            pl.BlockSpec((bm, 1), lambda i: (i, 0)),
            pl.BlockSpec((1, F), lambda i: (0, 0)),
        ],
        out_specs=pl.BlockSpec((bm, F), lambda i: (i, 0)),
        out_shape=jax.ShapeDtypeStruct((m, F), jnp.float32),
    )(s, g, deg.reshape(m, 1), b.reshape(1, F))


def _pool_body(x_ref, bat_ref, w_ref, sum_ref, max_ref, cnt_ref, sc_ref):
    i = pl.program_id(0)

    @pl.when(i == 0)
    def _():
        sum_ref[...] = jnp.zeros_like(sum_ref)
        max_ref[...] = jnp.full_like(max_ref, -jnp.inf)
        cnt_ref[...] = jnp.zeros_like(cnt_ref)

    x = x_ref[...]
    bat = bat_ref[...]
    wp = w_ref[...]
    nrm = jnp.sqrt(jnp.sum(wp * wp))
    sc_ref[...] = jnp.tanh(jnp.dot(x, wp, preferred_element_type=jnp.float32) / nrm)

    gi = lax.broadcasted_iota(jnp.int32, (x.shape[0], B), 1)
    oh = (bat == gi).astype(jnp.float32)
    sum_ref[...] += lax.dot_general(oh, x, (((0,), (0,)), ((), ())),
                                   preferred_element_type=jnp.float32)
    cnt_ref[...] += jnp.sum(oh, axis=0)[:, None]

    def body(g, _):
        mask = bat == g
        mx = jnp.max(jnp.where(mask, x, -jnp.inf), axis=0, keepdims=True)
        max_ref[pl.ds(g, 1), :] = jnp.maximum(max_ref[pl.ds(g, 1), :], mx)
        return 0
    lax.fori_loop(0, B, body, 0)

    @pl.when(i == pl.num_programs(0) - 1)
    def _():
        m = max_ref[...]
        max_ref[...] = jnp.where(jnp.isfinite(m), m, 0.0)


def _pools(x, bat, wp, bm=512):
    m = x.shape[0]
    return pl.pallas_call(
        _pool_body,
        grid=(m // bm,),
        in_specs=[
            pl.BlockSpec((bm, F), lambda i: (i, 0)),
            pl.BlockSpec((bm, 1), lambda i: (i, 0)),
            pl.BlockSpec((F, 128), lambda i: (0, 0)),
        ],
        out_specs=[
            pl.BlockSpec((B, F), lambda i: (0, 0)),
            pl.BlockSpec((B, F), lambda i: (0, 0)),
            pl.BlockSpec((B, 128), lambda i: (0, 0)),
            pl.BlockSpec((bm, 128), lambda i: (i, 0)),
        ],
        out_shape=[
            jax.ShapeDtypeStruct((B, F), jnp.float32),
            jax.ShapeDtypeStruct((B, F), jnp.float32),
            jax.ShapeDtypeStruct((B, 128), jnp.float32),
            jax.ShapeDtypeStruct((m, 128), jnp.float32),
        ],
    )(x, bat.reshape(m, 1), wp)


def _topk_body(s_ref, bat_ref, cnt_ref, ks_ref, kf_ref, b2_ref, t_s, t2_s, k_s, m_s):
    score = s_ref[...]
    bat = bat_ref[...]
    batf = bat.astype(jnp.float32)
    val = -(batf * 4.0 - score)
    u = lax.bitcast_convert_type(val, jnp.int32)
    sk = jnp.where(u < 0, u ^ np.int32(0x7FFFFFFF), u)
    msb = np.int32(-(2 ** 31))
    ss = sk ^ msb
    valid = bat < B

    def kinit(g, _):
        c = cnt_ref[pl.ds(g, 1), pl.ds(0, 1)][0, 0].astype(jnp.int32)
        k_s[g] = (4 * c + 4) // 5
        t_s[g] = 0
        t2_s[g] = 0
        return 0
    lax.fori_loop(0, B, kinit, 0)

    bits = [np.int32(-(2 ** 31))] + [np.int32(1 << b) for b in range(30, -1, -1)]
    for bit in bits:
        def gbody(g, _):
            cand = t_s[g] | bit
            scand = cand ^ msb
            ind = (ss >= scand) & (bat == g)
            cnt = jnp.sum(ind.astype(jnp.float32))
            t_s[g] = jnp.where(cnt >= k_s[g].astype(jnp.float32), cand, t_s[g])
            return 0
        lax.fori_loop(0, B, gbody, 0)

    def thr_body(g, acc):
        return jnp.where(bat == g, t_s[g], acc)
    thr = lax.fori_loop(0, B, thr_body, jnp.zeros_like(ss))
    sthr = thr ^ msb
    gt = ss > sthr
    tie = (ss == sthr) & valid

    def mb(g, _):
        ngt = jnp.sum((gt & (bat == g)).astype(jnp.float32))
        m_s[g] = k_s[g] - ngt.astype(jnp.int32)
        return 0
    lax.fori_loop(0, B, mb, 0)

    ridx = np.int32(16383) - (lax.broadcasted_iota(jnp.int32, score.shape, 0) * 128
                              + lax.broadcasted_iota(jnp.int32, score.shape, 1))
    for bit in [np.int32(1 << b) for b in range(13, -1, -1)]:
        def g2body(g, _):
            cand = t2_s[g] | bit
            ind = tie & (ridx >= cand) & (bat == g)
            cnt = jnp.sum(ind.astype(jnp.float32))
            t2_s[g] = jnp.where(cnt >= m_s[g].astype(jnp.float32), cand, t2_s[g])
            return 0
        lax.fori_loop(0, B, g2body, 0)

    def thr2_body(g, acc):
        return jnp.where(bat == g, t2_s[g], acc)
    thr2 = lax.fori_loop(0, B, thr2_body, jnp.zeros_like(ss))
    keep = (gt | (tie & (ridx >= thr2))) & valid
    keepf = keep.astype(jnp.float32)
    kf_ref[...] = keepf
    ks_ref[...] = keepf * score
    b2_ref[...] = jnp.where(keep, bat, B)


def _topk(s80, b80, cnt):
    return pl.pallas_call(
        _topk_body,
        grid=(1,),
        in_specs=[
            pl.BlockSpec((NP // 128, 128), lambda i: (0, 0)),
            pl.BlockSpec((NP // 128, 128), lambda i: (0, 0)),
            pl.BlockSpec((B, 128), lambda i: (0, 0)),
        ],
        out_specs=[
            pl.BlockSpec((NP // 128, 128), lambda i: (0, 0)),
            pl.BlockSpec((NP // 128, 128), lambda i: (0, 0)),
            pl.BlockSpec((NP // 128, 128), lambda i: (0, 0)),
        ],
        out_shape=[
            jax.ShapeDtypeStruct((NP // 128, 128), jnp.float32),
            jax.ShapeDtypeStruct((NP // 128, 128), jnp.float32),
            jax.ShapeDtypeStruct((NP // 128, 128), jnp.int32),
        ],
        scratch_shapes=[pltpu.SMEM((B,), jnp.int32)] * 4,
    )(s80, b80, cnt)


@functools.partial(
    pl.kernel, mesh=_mesh, compiler_params=_scparams,
    out_type=jax.ShapeDtypeStruct((32, SLAB), jnp.float32),
    scratch_types=[
        pltpu.VMEM((16,), jnp.int32),
        pltpu.VMEM((16,), jnp.int32),
        pltpu.VMEM((16 * C,), jnp.int32),
        pltpu.VMEM((16 * C,), jnp.int32),
        pltpu.VMEM((C, F), jnp.float32),
        pltpu.VMEM((C, F), jnp.float32),
        pltpu.VMEM((SLAB,), jnp.float32),
        pltpu.SemaphoreType.DMA,
        pltpu.SemaphoreType.DMA,
    ],
)
def _sc_conv(tab_hbm, row_hbm, col_hbm, bs_hbm, nc_hbm, out_hbm,
             bsv, ncv, rowblk, colblk, bufA, bufB, slab, semA, semB):
    tid = lax.axis_index("s")
    cid = lax.axis_index("c")
    IOTA = lax.iota(jnp.int32, 16)

    pltpu.sync_copy(bs_hbm.at[cid], bsv)
    pltpu.sync_copy(nc_hbm.at[cid], ncv)
    lo = jnp.sum(jnp.where(IOTA == tid, bsv[...], 0))
    nch = jnp.sum(jnp.where(IOTA == tid, ncv[...], 0))

    zero = jnp.zeros((16,), jnp.float32)

    def zbody(i, _):
        for j in range(8):
            slab[pl.ds(i * 128 + j * 16, 16)] = zero
        return 0
    lax.fori_loop(0, SLAB // 128, zbody, 0)

    IB = 16

    def start(c, buf, sem):
        pltpu.async_copy(tab_hbm.at[rowblk.at[pl.ds(c * C, C)]], buf, sem)

    def process(c, buf, sem):
        pltpu.make_async_copy(tab_hbm.at[rowblk.at[pl.ds(c * C, C)]], buf,
                              sem).wait()

        def ebody(e, _):
            for u in range(2):
                ei = c * C + e * 2 + u
                cvec = plsc.load_gather(colblk, [jnp.full((16,), ei, jnp.int32)])
                a0 = cvec * F + IOTA
                for kk in range(16):
                    dat = buf[e * 2 + u, pl.ds(kk * 16, 16)]
                    plsc.addupdate_scatter(slab, [a0 + (kk * 16)], dat)
            return 0
        lax.fori_loop(0, C // 2, ebody, 0)

    def blk_body(blk, _):
        cbase = blk * IB
        base = pl.multiple_of(lo + cbase * C, C)
        pltpu.sync_copy(row_hbm.at[pl.ds(base, IB * C)], rowblk)
        pltpu.sync_copy(col_hbm.at[pl.ds(base, IB * C)], colblk)

        @pl.when(cbase < nch)
        def _():
            start(0, bufA, semA)
        for c in range(IB):
            buf, sem = (bufA, semA) if c % 2 == 0 else (bufB, semB)
            nbuf, nsem = (bufB, semB) if c % 2 == 0 else (bufA, semA)

            @pl.when(cbase + c < nch)
            def _():
                if c + 1 < IB:
                    @pl.when(cbase + c + 1 < nch)
                    def _():
                        start(c + 1, nbuf, nsem)
                process(c, buf, sem)
        return 0

    lax.fori_loop(0, (nch + IB - 1) // IB, blk_body, 0)
    pltpu.sync_copy(slab, out_hbm.at[cid * 16 + tid])


@functools.partial(
    pl.kernel, mesh=_mesh, compiler_params=_scparams,
    out_type=jax.ShapeDtypeStruct((32, 320), jnp.float32),
    scratch_types=[
        pltpu.VMEM((16,), jnp.int32),
        pltpu.VMEM((16,), jnp.int32),
        pltpu.VMEM((10016,), jnp.float32),
        pltpu.VMEM((C,), jnp.int32),
        pltpu.VMEM((C,), jnp.int32),
        pltpu.VMEM((320,), jnp.float32),
    ],
)
def _sc_deg(keep_hbm, row_hbm, col_hbm, bs_hbm, nc_hbm, out_hbm,
            bsv, ncv, keepv, rowv, colv, slab):
    tid = lax.axis_index("s")
    cid = lax.axis_index("c")
    IOTA = lax.iota(jnp.int32, 16)

    pltpu.sync_copy(bs_hbm.at[cid], bsv)
    pltpu.sync_copy(nc_hbm.at[cid], ncv)
    pltpu.sync_copy(keep_hbm, keepv)
    lo = jnp.sum(jnp.where(IOTA == tid, bsv[...], 0))
    nch = jnp.sum(jnp.where(IOTA == tid, ncv[...], 0))
    zero = jnp.zeros((16,), jnp.float32)
    for i in range(320 // 16):
        slab[pl.ds(i * 16, 16)] = zero

    def body(i, _):
        base = pl.multiple_of(lo + i * C, C)
        pltpu.sync_copy(row_hbm.at[pl.ds(base, C)], rowv)
        pltpu.sync_copy(col_hbm.at[pl.ds(base, C)], colv)

        def ebody(j, _):
            for u in range(2):
                jj = (j * 2 + u) * 16
                kv = plsc.load_gather(keepv, [rowv[pl.ds(jj, 16)]])
                plsc.addupdate_scatter(slab, [colv[pl.ds(jj, 16)]], kv)
            return 0
        lax.fori_loop(0, C // 32, ebody, 0)
        return 0

    lax.fori_loop(0, nch, body, 0)
    pltpu.sync_copy(slab, out_hbm.at[cid * 16 + tid])


def _pad_rows(a, rows=NP):
    return jnp.pad(a, ((0, rows - a.shape[0]),) + ((0, 0),) * (a.ndim - 1))


def _mk_table(g):
    z = jnp.zeros((8, F), jnp.float32)
    return jnp.concatenate([g[:N], z], axis=0)


def _conv(tab, row_s, colrel_s, bstart, nch16):
    out = _sc_conv(tab, row_s, colrel_s, bstart, nch16)
    return out.reshape(NB * BASE, F)[:N]


def kernel(x, params, edge_index, batch):
    p = params
    row = edge_index[0]
    col = edge_index[1]

    b = col // BASE
    cnt16 = jnp.bincount(b, length=NB)
    cap = ((cnt16 + C - 1) // C) * C
    pstart = jnp.concatenate([jnp.zeros((1,), jnp.int32),
                              jnp.cumsum(cap)[:-1].astype(jnp.int32)])
    nch16 = (cap // C).astype(jnp.int32)
    oh16 = (b[:, None] == jnp.arange(NB, dtype=b.dtype))
    rank = jnp.take_along_axis(jnp.cumsum(oh16.astype(jnp.int32), axis=0),
                               b[:, None].astype(jnp.int32), axis=1)[:, 0] - 1
    pos = pstart[b] + rank
    row_s = jnp.full((EP,), DUMMY, jnp.int32).at[pos].set(row)
    colrel_s = jnp.zeros((EP,), jnp.int32).at[pos].set(col - b * BASE)
    bstart = pstart.astype(jnp.int32).reshape(2, 16)
    nch16 = nch16.reshape(2, 16)

    xp = _pad_rows(x)
    batp = jnp.concatenate([batch, jnp.full((NP - N,), B, jnp.int32)])

    onestab = jnp.concatenate([jnp.ones((N,), jnp.float32),
                               jnp.zeros((16,), jnp.float32)])
    dcnt1 = _sc_deg(onestab, row_s, colrel_s, bstart, nch16)
    deg1 = dcnt1[:, :BASE].reshape(NB * BASE)[:N] + 1.0
    deg1p = _pad_rows(deg1[:, None], NP)[:, 0] + jnp.where(
        jnp.arange(NP) < N, 0.0, 1.0)

    g1 = _mm(xp, p['b1_c1_W'], jnp.zeros((F,), jnp.float32), deg=deg1p)
    s1 = _conv(_mk_table(g1), row_s, colrel_s, bstart, nch16)
    x1 = _gcn_post(_pad_rows(s1), g1, deg1p, p['b1_c1_b'])
    g2 = _mm(x1, p['b1_c2_W'], jnp.zeros((F,), jnp.float32), deg=deg1p)
    s2 = _conv(_mk_table(g2), row_s, colrel_s, bstart, nch16)
    x2 = _gcn_post(_pad_rows(s2), g2, deg1p, p['b1_c2_b'])
    xb1 = _mm(jnp.concatenate([x1, x2], axis=1), p['b1_lin_W'], p['b1_lin_b'],
              act="relu")

    wp = jnp.zeros((F, 128), jnp.float32).at[:, 0].set(p['pool_p'])
    sum1, max1, cnt, sc2d = _pools(xb1, batp, wp)
    score80 = sc2d[:, 0].reshape(NP // 128, 128)
    b80 = batp.reshape(NP // 128, 128)

    ks80, kf80, b2_80 = _topk(score80, b80, cnt)
    kscore = ks80.reshape(NP)
    keepf = kf80.reshape(NP)
    bat2 = b2_80.reshape(NP)

    keeptab = jnp.concatenate([keepf[:N], jnp.zeros((16,), jnp.float32)])
    dcnt = _sc_deg(keeptab, row_s, colrel_s, bstart, nch16)
    deg2 = dcnt[:, :BASE].reshape(NB * BASE)[:N] + 1.0
    deg2p = _pad_rows(deg2[:, None], NP)[:, 0] + jnp.where(
        jnp.arange(NP) < N, 0.0, 1.0)

    g3 = _mm(xb1, p['b2_c1_W'], jnp.zeros((F,), jnp.float32),
             pre=kscore, deg=deg2p)
    s3 = _conv(_mk_table(g3), row_s, colrel_s, bstart, nch16)
    x1b = _gcn_post(_pad_rows(s3), g3, deg2p, p['b2_c1_b'])
    g4 = _mm(x1b, p['b2_c2_W'], jnp.zeros((F,), jnp.float32),
             pre=keepf, deg=deg2p)
    s4 = _conv(_mk_table(g4), row_s, colrel_s, bstart, nch16)
    x2b = _gcn_post(_pad_rows(s4), g4, deg2p, p['b2_c2_b'])
    xb2 = _mm(jnp.concatenate([x1b, x2b], axis=1), p['b2_lin_W'], p['b2_lin_b'],
              act="relu")

    sum2, max2, _, _ = _pools(xb2, bat2, wp)

    h = jnp.concatenate([sum1, max1, sum2, max2], axis=1)
    h = _mm(h, p['lin1_W'], p['lin1_b'], act="relu", bm=B)
    w2p = jnp.pad(p['lin2_W'], ((0, 0), (0, 118)))
    b2p = jnp.pad(p['lin2_b'], (0, 118))
    out = _mm(h, w2p, b2p, act="softmax", bm=B)
    return out[:, :10]
